# baseline (device time: 50202 ns/iter reference)
import functools

import jax
import jax.numpy as jnp
from jax import lax
from jax.experimental import pallas as pl
from jax.experimental.pallas import tpu as pltpu

N_DEV = 4
SQ = 256
SKV = 4096
D = 1024
DH = 128
H_LOCAL = 8
SCALE = 0.08838834764831843


def kernel(x, Wq, Wo, K_ext, V_ext):
    my = lax.axis_index("i")

    xb = x[0].astype(jnp.bfloat16)
    Wqb = Wq.astype(jnp.bfloat16)
    Wob = Wo.astype(jnp.bfloat16)
    K2 = lax.dynamic_slice_in_dim(K_ext[0], 2 * my, 2, axis=1)
    V2 = lax.dynamic_slice_in_dim(V_ext[0], 2 * my, 2, axis=1)
    Kl = jnp.transpose(K2, (1, 0, 2)).astype(jnp.bfloat16)
    Vl = jnp.transpose(V2, (1, 0, 2)).astype(jnp.bfloat16)

    def body(x_ref, wq_ref, wo_ref, k_ref, v_ref, out_ref,
             send_buf, comm, send_sems, recv_sems):
        my_pos = lax.axis_index("i")

        barrier = pltpu.get_barrier_semaphore()
        for d in range(1, N_DEV):
            pl.semaphore_signal(
                barrier, inc=1,
                device_id=((my_pos + d) % N_DEV,),
                device_id_type=pl.DeviceIdType.MESH,
            )
        pl.semaphore_wait(barrier, N_DEV - 1)

        q = jnp.dot(x_ref[...], wq_ref[...],
                    preferred_element_type=jnp.float32)
        q = (q * SCALE).astype(jnp.bfloat16)

        outs = []
        for h in range(H_LOCAL):
            qh = q[:, h * DH:(h + 1) * DH]
            kh = k_ref[h // 4]
            vh = v_ref[h // 4]
            s = lax.dot_general(
                qh, kh, (((1,), (1,)), ((), ())),
                preferred_element_type=jnp.float32)
            m = jnp.max(s, axis=1, keepdims=True)
            p = jnp.exp(s - m)
            l = jnp.sum(p, axis=1, keepdims=True)
            o = jnp.dot(p.astype(jnp.bfloat16), vh,
                        preferred_element_type=jnp.float32) / l
            outs.append(o.astype(jnp.bfloat16))
        attn = jnp.concatenate(outs, axis=1)

        partial = jnp.dot(attn, wo_ref[...],
                          preferred_element_type=jnp.float32)
        send_buf[...] = partial.astype(jnp.bfloat16)

        rdmas = []
        for d in range(1, N_DEV):
            r = pltpu.make_async_remote_copy(
                src_ref=send_buf,
                dst_ref=comm.at[d - 1],
                send_sem=send_sems.at[d - 1],
                recv_sem=recv_sems.at[d - 1],
                device_id=((my_pos + d) % N_DEV,),
                device_id_type=pl.DeviceIdType.MESH,
            )
            r.start()
            rdmas.append(r)

        acc = partial
        for d in range(1, N_DEV):
            rdmas[d - 1].wait_recv()
            acc = acc + comm[d - 1].astype(jnp.float32)
        out_ref[...] = acc
        for r in rdmas:
            r.wait_send()

        @functools.partial(pl.run_scoped, exit_sem=pltpu.SemaphoreType.REGULAR)
        def _(exit_sem):
            for d in range(1, N_DEV):
                pl.semaphore_signal(
                    exit_sem, inc=1,
                    device_id=((my_pos + d) % N_DEV,),
                    device_id_type=pl.DeviceIdType.MESH,
                )
            pl.semaphore_wait(exit_sem, N_DEV - 1)

    out = pl.pallas_call(
        body,
        out_shape=jax.ShapeDtypeStruct((SQ, D), jnp.float32),
        in_specs=[pl.BlockSpec(memory_space=pltpu.VMEM)] * 5,
        out_specs=pl.BlockSpec(memory_space=pltpu.VMEM),
        scratch_shapes=[
            pltpu.VMEM((SQ, D), jnp.bfloat16),
            pltpu.VMEM((N_DEV - 1, SQ, D), jnp.bfloat16),
            pltpu.SemaphoreType.DMA((N_DEV - 1,)),
            pltpu.SemaphoreType.DMA((N_DEV - 1,)),
        ],
        compiler_params=pltpu.CompilerParams(collective_id=0),
    )(xb, Wqb, Wob, Kl, Vl)
    return out[None]
